# baseline (speedup 1.0000x reference)
import functools

import jax
import jax.numpy as jnp
from jax import lax
from jax.experimental import pallas as pl
from jax.experimental.pallas import tpu as pltpu
from jax.experimental.pallas import tpu_sc as plsc

_INTERPRET = False

T = 1024
NW = 32
L = 16
RCHUNK = 64


def _pos_body(ch_ref, pos_ref, counts_ref):
    E = 16
    G, B = ch_ref.shape
    chm = ch_ref[...] % E
    tri_b = (jax.lax.broadcasted_iota(jnp.int32, (B, B), 0)
             < jax.lax.broadcasted_iota(jnp.int32, (B, B), 1)).astype(jnp.float32)
    tri_g = (jax.lax.broadcasted_iota(jnp.int32, (G, G), 1)
             < jax.lax.broadcasted_iota(jnp.int32, (G, G), 0)).astype(jnp.float32)
    iota_e = jax.lax.broadcasted_iota(jnp.int32, (1, E), 1)
    pos = jnp.zeros((G, B), jnp.float32)
    counts = jnp.zeros((1, E), jnp.int32)
    run_tot = jnp.float32(0.0)
    for e in range(E):
        m = (chm == e).astype(jnp.float32)
        rank = jnp.dot(m, tri_b, preferred_element_type=jnp.float32)
        cnt = jnp.sum(m, axis=1, keepdims=True)
        row_base = jnp.dot(tri_g, cnt, preferred_element_type=jnp.float32)
        tot = jnp.sum(cnt)
        pos = pos + m * (run_tot + row_base + rank)
        counts = jnp.where(iota_e == e, tot.astype(jnp.int32), counts)
        run_tot = run_tot + tot
    pos_ref[...] = pos.astype(jnp.int32)
    counts_ref[...] = counts


def _pos_tc(ch2d):
    G, B = ch2d.shape
    return pl.pallas_call(
        _pos_body,
        out_shape=[jax.ShapeDtypeStruct((G, B), jnp.int32),
                   jax.ShapeDtypeStruct((1, 16), jnp.int32)],
        interpret=_INTERPRET,
    )(ch2d)


def _shuffle_sc(xv, posm):
    N, DW = xv.shape
    seg = N // NW
    mesh = plsc.VectorSubcoreMesh(core_axis_name="c", subcore_axis_name="s")

    @functools.partial(
        pl.kernel, mesh=mesh,
        out_type=jax.ShapeDtypeStruct((N, DW), jnp.float32),
        scratch_types=[
            pltpu.VMEM((seg // RCHUNK, RCHUNK), jnp.int32),
            pltpu.VMEM((RCHUNK, DW), jnp.float32),
            pltpu.SemaphoreType.DMA,
        ],
    )
    def shuffle(xv_hbm, pos_hbm, xs_hbm, posm_v, rows_v, sem):
        w = lax.axis_index("s") * 2 + lax.axis_index("c")
        pltpu.sync_copy(pos_hbm.at[w], posm_v)
        for c in range(seg // RCHUNK):
            pltpu.sync_copy(xv_hbm.at[pl.ds(w * seg + c * RCHUNK, RCHUNK)], rows_v)
            pltpu.async_copy(rows_v, xs_hbm.at[posm_v.at[c]], sem).wait()

    return shuffle(xv, posm)


def _unsort_sc(prob_sorted, pos):
    N = prob_sorted.shape[0]
    seg = N // NW
    mesh = plsc.VectorSubcoreMesh(core_axis_name="c", subcore_axis_name="s")

    @functools.partial(
        pl.kernel, mesh=mesh,
        out_type=jax.ShapeDtypeStruct((NW, seg), jnp.float32),
        scratch_types=[
            pltpu.VMEM((seg,), jnp.int32),
            pltpu.VMEM((seg,), jnp.float32),
            pltpu.SemaphoreType.DMA,
        ],
    )
    def unsort(prob_hbm, pos_hbm, out_hbm, pos_v, vals_v, sem):
        w = lax.axis_index("s") * 2 + lax.axis_index("c")
        pltpu.sync_copy(pos_hbm.at[w], pos_v)
        pltpu.async_copy(prob_hbm.at[pos_v], vals_v, sem).wait()
        pltpu.sync_copy(vals_v, out_hbm.at[w])

    return unsort(prob_sorted, pos).reshape(N)


def _gmm_body(tile_ids, group_ids, offs, b2s,
              x_ref, w1_ref, b1_ref, w2_ref, out_ref):
    u = pl.program_id(0)
    e = group_ids[u]
    t = tile_ids[u]
    h = jnp.tanh(
        jnp.dot(x_ref[...].astype(jnp.bfloat16),
                w1_ref[0].astype(jnp.bfloat16),
                preferred_element_type=jnp.float32)
        + b1_ref[0, 0, :][None, :])
    logit = jax.lax.dot_general(
        h, w2_ref[0], (((1,), (0,)), ((), ())),
        preferred_element_type=jnp.float32) + b2s[e]
    prob = jax.nn.softplus(logit)
    r = t * T + jax.lax.broadcasted_iota(jnp.int32, (T, 1), 0)
    mask = (r >= offs[e]) & (r < offs[e + 1])
    out_ref[...] = jnp.where(mask, prob, out_ref[...])


def _grouped_mlp(xs, W1, b1, w2, b2, tile_ids, group_ids, offs):
    N, D = xs.shape
    E, _, F = W1.shape
    U = tile_ids.shape[0]
    grid_spec = pltpu.PrefetchScalarGridSpec(
        num_scalar_prefetch=4,
        grid=(U,),
        in_specs=[
            pl.BlockSpec((T, D), lambda u, ti, gi, of, b2s: (ti[u], 0)),
            pl.BlockSpec((1, D, F), lambda u, ti, gi, of, b2s: (gi[u], 0, 0)),
            pl.BlockSpec((1, 1, F), lambda u, ti, gi, of, b2s: (gi[u], 0, 0)),
            pl.BlockSpec((1, F, 1), lambda u, ti, gi, of, b2s: (gi[u], 0, 0)),
        ],
        out_specs=pl.BlockSpec((T, 1), lambda u, ti, gi, of, b2s: (ti[u], 0)),
    )
    out = pl.pallas_call(
        _gmm_body,
        grid_spec=grid_spec,
        out_shape=jax.ShapeDtypeStruct((N, 1), jnp.float32),
        interpret=_INTERPRET,
    )(tile_ids, group_ids, offs, b2,
      xs, W1, b1.reshape(E, 1, F), w2.reshape(E, F, 1))
    return out.reshape(N)


def _metadata(counts, num_tiles):
    E = counts.shape[0]
    U = num_tiles + E - 1
    ends = jnp.cumsum(counts)
    starts = ends - counts
    offs = jnp.concatenate([jnp.zeros((1,), jnp.int32), ends.astype(jnp.int32)])
    start_tile = starts // T
    end_tile = jnp.maximum(ends - 1, 0) // T
    g = jnp.where(counts > 0, end_tile - start_tile + 1, 0)
    gcum = jnp.cumsum(g)
    total = gcum[-1]
    u = jnp.arange(U, dtype=jnp.int32)
    uc = jnp.minimum(u, total - 1)
    grp = jnp.searchsorted(gcum, uc, side="right").astype(jnp.int32)
    before = (gcum[grp] - g[grp]).astype(jnp.int32)
    tile = (start_tile[grp] + (uc - before)).astype(jnp.int32)
    return tile, grp, offs


def kernel(x, channel, W1, b1, w2, b2):
    N, D = x.shape
    E = W1.shape[0]
    seg = N // NW
    pos, counts = _pos_tc(channel.astype(jnp.int32).reshape(N // 128, 128))
    posm = pos.reshape(NW, seg // RCHUNK, RCHUNK)
    xs = _shuffle_sc(x, posm)
    tile_ids, group_ids, offs = _metadata(counts.reshape(E), N // T)
    prob_sorted = _grouped_mlp(xs, W1, b1, w2, b2, tile_ids, group_ids, offs)
    return _unsort_sc(prob_sorted, pos.reshape(NW, seg))


# speedup vs baseline: 1.1774x; 1.1774x over previous
import functools

import jax
import jax.numpy as jnp
from jax import lax
from jax.experimental import pallas as pl
from jax.experimental.pallas import tpu as pltpu
from jax.experimental.pallas import tpu_sc as plsc

_INTERPRET = False

T = 512
NW = 32
L = 16
RCHUNK = 64


def _pos_body(ch_ref, pos_ref, counts_ref):
    E = 16
    G, B = ch_ref.shape
    chm = ch_ref[...] % E
    tri_b = (jax.lax.broadcasted_iota(jnp.int32, (B, B), 0)
             < jax.lax.broadcasted_iota(jnp.int32, (B, B), 1)).astype(jnp.float32)
    tri_g = (jax.lax.broadcasted_iota(jnp.int32, (G, G), 1)
             < jax.lax.broadcasted_iota(jnp.int32, (G, G), 0)).astype(jnp.float32)
    iota_e = jax.lax.broadcasted_iota(jnp.int32, (1, E), 1)
    pos = jnp.zeros((G, B), jnp.float32)
    counts = jnp.zeros((1, E), jnp.int32)
    run_tot = jnp.float32(0.0)
    for e in range(E):
        m = (chm == e).astype(jnp.float32)
        rank = jnp.dot(m, tri_b, preferred_element_type=jnp.float32)
        cnt = jnp.sum(m, axis=1, keepdims=True)
        row_base = jnp.dot(tri_g, cnt, preferred_element_type=jnp.float32)
        tot = jnp.sum(cnt)
        pos = pos + m * (run_tot + row_base + rank)
        counts = jnp.where(iota_e == e, tot.astype(jnp.int32), counts)
        run_tot = run_tot + tot
    pos_ref[...] = pos.astype(jnp.int32)
    counts_ref[...] = counts


def _pos_tc(ch2d):
    G, B = ch2d.shape
    return pl.pallas_call(
        _pos_body,
        out_shape=[jax.ShapeDtypeStruct((G, B), jnp.int32),
                   jax.ShapeDtypeStruct((1, 16), jnp.int32)],
        interpret=_INTERPRET,
    )(ch2d)


def _shuffle_sc(xv, posm):
    N, DW = xv.shape
    seg = N // NW
    mesh = plsc.VectorSubcoreMesh(core_axis_name="c", subcore_axis_name="s")

    @functools.partial(
        pl.kernel, mesh=mesh,
        out_type=jax.ShapeDtypeStruct((N, DW), jnp.float32),
        scratch_types=[
            pltpu.VMEM((seg // RCHUNK, RCHUNK), jnp.int32),
            pltpu.VMEM((RCHUNK, DW), jnp.float32),
            pltpu.SemaphoreType.DMA,
        ],
    )
    def shuffle(xv_hbm, pos_hbm, xs_hbm, posm_v, rows_v, sem):
        w = lax.axis_index("s") * 2 + lax.axis_index("c")
        pltpu.sync_copy(pos_hbm.at[w], posm_v)
        for c in range(seg // RCHUNK):
            pltpu.sync_copy(xv_hbm.at[pl.ds(w * seg + c * RCHUNK, RCHUNK)], rows_v)
            pltpu.async_copy(rows_v, xs_hbm.at[posm_v.at[c]], sem).wait()

    return shuffle(xv, posm)


def _unsort_sc(prob_sorted, pos):
    N = prob_sorted.shape[0]
    seg = N // NW
    mesh = plsc.VectorSubcoreMesh(core_axis_name="c", subcore_axis_name="s")

    @functools.partial(
        pl.kernel, mesh=mesh,
        out_type=jax.ShapeDtypeStruct((NW, seg), jnp.float32),
        scratch_types=[
            pltpu.VMEM((seg,), jnp.int32),
            pltpu.VMEM((seg,), jnp.float32),
            pltpu.SemaphoreType.DMA,
        ],
    )
    def unsort(prob_hbm, pos_hbm, out_hbm, pos_v, vals_v, sem):
        w = lax.axis_index("s") * 2 + lax.axis_index("c")
        pltpu.sync_copy(pos_hbm.at[w], pos_v)
        pltpu.async_copy(prob_hbm.at[pos_v], vals_v, sem).wait()
        pltpu.sync_copy(vals_v, out_hbm.at[w])

    return unsort(prob_sorted, pos).reshape(N)


def _gmm_body(tile_ids, group_ids, offs, b2s,
              x_ref, w1_ref, b1_ref, w2_ref, out_ref):
    u = pl.program_id(0)
    e = group_ids[u]
    t = tile_ids[u]
    h = jnp.tanh(
        jnp.dot(x_ref[...].astype(jnp.bfloat16),
                w1_ref[0].astype(jnp.bfloat16),
                preferred_element_type=jnp.float32)
        + b1_ref[0, 0, :][None, :])
    logit = jax.lax.dot_general(
        h, w2_ref[0], (((1,), (0,)), ((), ())),
        preferred_element_type=jnp.float32) + b2s[e]
    prob = jax.nn.softplus(logit)
    r = t * T + jax.lax.broadcasted_iota(jnp.int32, (T, 1), 0)
    mask = (r >= offs[e]) & (r < offs[e + 1])
    out_ref[...] = jnp.where(mask, prob, out_ref[...])


def _grouped_mlp(xs, W1, b1, w2, b2, tile_ids, group_ids, offs):
    N, D = xs.shape
    E, _, F = W1.shape
    U = tile_ids.shape[0]
    grid_spec = pltpu.PrefetchScalarGridSpec(
        num_scalar_prefetch=4,
        grid=(U,),
        in_specs=[
            pl.BlockSpec((T, D), lambda u, ti, gi, of, b2s: (ti[u], 0)),
            pl.BlockSpec((1, D, F), lambda u, ti, gi, of, b2s: (gi[u], 0, 0)),
            pl.BlockSpec((1, 1, F), lambda u, ti, gi, of, b2s: (gi[u], 0, 0)),
            pl.BlockSpec((1, F, 1), lambda u, ti, gi, of, b2s: (gi[u], 0, 0)),
        ],
        out_specs=pl.BlockSpec((T, 1), lambda u, ti, gi, of, b2s: (ti[u], 0)),
    )
    out = pl.pallas_call(
        _gmm_body,
        grid_spec=grid_spec,
        out_shape=jax.ShapeDtypeStruct((N, 1), jnp.float32),
        interpret=_INTERPRET,
    )(tile_ids, group_ids, offs, b2,
      xs, W1, b1.reshape(E, 1, F), w2.reshape(E, F, 1))
    return out.reshape(N)


def _metadata(counts, num_tiles):
    E = counts.shape[0]
    U = num_tiles + E - 1
    ends = jnp.cumsum(counts)
    starts = ends - counts
    offs = jnp.concatenate([jnp.zeros((1,), jnp.int32), ends.astype(jnp.int32)])
    start_tile = starts // T
    end_tile = jnp.maximum(ends - 1, 0) // T
    g = jnp.where(counts > 0, end_tile - start_tile + 1, 0)
    gcum = jnp.cumsum(g)
    total = gcum[-1]
    u = jnp.arange(U, dtype=jnp.int32)
    uc = jnp.minimum(u, total - 1)
    grp = jnp.searchsorted(gcum, uc, side="right").astype(jnp.int32)
    before = (gcum[grp] - g[grp]).astype(jnp.int32)
    tile = (start_tile[grp] + (uc - before)).astype(jnp.int32)
    return tile, grp, offs


def kernel(x, channel, W1, b1, w2, b2):
    N, D = x.shape
    E = W1.shape[0]
    seg = N // NW
    pos, counts = _pos_tc(channel.astype(jnp.int32).reshape(N // 128, 128))
    posm = pos.reshape(NW, seg // RCHUNK, RCHUNK)
    xs = _shuffle_sc(x, posm)
    tile_ids, group_ids, offs = _metadata(counts.reshape(E), N // T)
    prob_sorted = _grouped_mlp(xs, W1, b1, w2, b2, tile_ids, group_ids, offs)
    return _unsort_sc(prob_sorted, pos.reshape(NW, seg))


# speedup vs baseline: 1.1877x; 1.0087x over previous
import functools

import jax
import jax.numpy as jnp
from jax import lax
from jax.experimental import pallas as pl
from jax.experimental.pallas import tpu as pltpu
from jax.experimental.pallas import tpu_sc as plsc

_INTERPRET = False

T = 512
NW = 32
L = 16
RCHUNK = 64


def _pos_body(ch_ref, pos_ref, counts_ref):
    E = 16
    G, B = ch_ref.shape
    chm = ch_ref[...] % E
    tri_b = (jax.lax.broadcasted_iota(jnp.int32, (B, B), 0)
             < jax.lax.broadcasted_iota(jnp.int32, (B, B), 1)).astype(jnp.float32)
    tri_g = (jax.lax.broadcasted_iota(jnp.int32, (G, G), 1)
             < jax.lax.broadcasted_iota(jnp.int32, (G, G), 0)).astype(jnp.float32)
    iota_e = jax.lax.broadcasted_iota(jnp.int32, (1, E), 1)
    pos = jnp.zeros((G, B), jnp.float32)
    counts = jnp.zeros((1, E), jnp.int32)
    run_tot = jnp.float32(0.0)
    for e in range(E):
        m = (chm == e).astype(jnp.float32)
        rank = jnp.dot(m, tri_b, preferred_element_type=jnp.float32)
        cnt = jnp.sum(m, axis=1, keepdims=True)
        row_base = jnp.dot(tri_g, cnt, preferred_element_type=jnp.float32)
        tot = jnp.sum(cnt)
        pos = pos + m * (run_tot + row_base + rank)
        counts = jnp.where(iota_e == e, tot.astype(jnp.int32), counts)
        run_tot = run_tot + tot
    pos_ref[...] = pos.astype(jnp.int32)
    counts_ref[...] = counts


def _pos_tc(ch2d):
    G, B = ch2d.shape
    return pl.pallas_call(
        _pos_body,
        out_shape=[jax.ShapeDtypeStruct((G, B), jnp.int32),
                   jax.ShapeDtypeStruct((1, 16), jnp.int32)],
        interpret=_INTERPRET,
    )(ch2d)


def _shuffle_sc(xv, posm):
    N, DW = xv.shape
    seg = N // NW
    mesh = plsc.VectorSubcoreMesh(core_axis_name="c", subcore_axis_name="s")

    @functools.partial(
        pl.kernel, mesh=mesh,
        out_type=jax.ShapeDtypeStruct((N, DW), jnp.float32),
        scratch_types=[
            pltpu.VMEM((seg // RCHUNK, RCHUNK), jnp.int32),
            pltpu.VMEM((RCHUNK, DW), jnp.float32),
            pltpu.SemaphoreType.DMA,
        ],
    )
    def shuffle(xv_hbm, pos_hbm, xs_hbm, posm_v, rows_v, sem):
        w = lax.axis_index("s") * 2 + lax.axis_index("c")
        pltpu.sync_copy(pos_hbm.at[w], posm_v)
        for c in range(seg // RCHUNK):
            pltpu.sync_copy(xv_hbm.at[pl.ds(w * seg + c * RCHUNK, RCHUNK)], rows_v)
            pltpu.async_copy(rows_v, xs_hbm.at[posm_v.at[c]], sem).wait()

    return shuffle(xv, posm)


def _unsort_sc(prob_sorted, pos):
    N = prob_sorted.shape[0]
    seg = N // NW
    mesh = plsc.VectorSubcoreMesh(core_axis_name="c", subcore_axis_name="s")

    @functools.partial(
        pl.kernel, mesh=mesh,
        out_type=jax.ShapeDtypeStruct((NW, seg), jnp.float32),
        scratch_types=[
            pltpu.VMEM((seg,), jnp.int32),
            pltpu.VMEM((seg,), jnp.float32),
            pltpu.SemaphoreType.DMA,
        ],
    )
    def unsort(prob_hbm, pos_hbm, out_hbm, pos_v, vals_v, sem):
        w = lax.axis_index("s") * 2 + lax.axis_index("c")
        pltpu.sync_copy(pos_hbm.at[w], pos_v)
        pltpu.async_copy(prob_hbm.at[pos_v], vals_v, sem).wait()
        pltpu.sync_copy(vals_v, out_hbm.at[w])

    return unsort(prob_sorted, pos).reshape(N)


def _gmm_body(tile_ids, group_ids, offs, b2s,
              x_ref, w1_ref, b1_ref, w2_ref, out_ref):
    u = pl.program_id(0)
    e = group_ids[u]
    t = tile_ids[u]
    w1 = w1_ref[0].astype(jnp.bfloat16)
    b1r = b1_ref[0, 0, :][None, :]

    def _half(sl):
        h = jnp.tanh(
            jnp.dot(x_ref[sl, :].astype(jnp.bfloat16), w1,
                    preferred_element_type=jnp.float32) + b1r)
        return jax.lax.dot_general(
            h, w2_ref[0], (((1,), (0,)), ((), ())),
            preferred_element_type=jnp.float32)

    H = T // 2
    logit = jnp.concatenate(
        [_half(pl.ds(0, H)), _half(pl.ds(H, H))], axis=0) + b2s[e]
    prob = jax.nn.softplus(logit)
    r = t * T + jax.lax.broadcasted_iota(jnp.int32, (T, 1), 0)
    mask = (r >= offs[e]) & (r < offs[e + 1])
    out_ref[...] = jnp.where(mask, prob, out_ref[...])


def _grouped_mlp(xs, W1, b1, w2, b2, tile_ids, group_ids, offs):
    N, D = xs.shape
    E, _, F = W1.shape
    U = tile_ids.shape[0]
    grid_spec = pltpu.PrefetchScalarGridSpec(
        num_scalar_prefetch=4,
        grid=(U,),
        in_specs=[
            pl.BlockSpec((T, D), lambda u, ti, gi, of, b2s: (ti[u], 0)),
            pl.BlockSpec((1, D, F), lambda u, ti, gi, of, b2s: (gi[u], 0, 0)),
            pl.BlockSpec((1, 1, F), lambda u, ti, gi, of, b2s: (gi[u], 0, 0)),
            pl.BlockSpec((1, F, 1), lambda u, ti, gi, of, b2s: (gi[u], 0, 0)),
        ],
        out_specs=pl.BlockSpec((T, 1), lambda u, ti, gi, of, b2s: (ti[u], 0)),
    )
    out = pl.pallas_call(
        _gmm_body,
        grid_spec=grid_spec,
        out_shape=jax.ShapeDtypeStruct((N, 1), jnp.float32),
        interpret=_INTERPRET,
    )(tile_ids, group_ids, offs, b2,
      xs, W1, b1.reshape(E, 1, F), w2.reshape(E, F, 1))
    return out.reshape(N)


def _metadata(counts, num_tiles):
    E = counts.shape[0]
    U = num_tiles + E - 1
    ends = jnp.cumsum(counts)
    starts = ends - counts
    offs = jnp.concatenate([jnp.zeros((1,), jnp.int32), ends.astype(jnp.int32)])
    start_tile = starts // T
    end_tile = jnp.maximum(ends - 1, 0) // T
    g = jnp.where(counts > 0, end_tile - start_tile + 1, 0)
    gcum = jnp.cumsum(g)
    total = gcum[-1]
    u = jnp.arange(U, dtype=jnp.int32)
    uc = jnp.minimum(u, total - 1)
    grp = jnp.searchsorted(gcum, uc, side="right").astype(jnp.int32)
    before = (gcum[grp] - g[grp]).astype(jnp.int32)
    tile = (start_tile[grp] + (uc - before)).astype(jnp.int32)
    return tile, grp, offs


def kernel(x, channel, W1, b1, w2, b2):
    N, D = x.shape
    E = W1.shape[0]
    seg = N // NW
    pos, counts = _pos_tc(channel.astype(jnp.int32).reshape(N // 128, 128))
    posm = pos.reshape(NW, seg // RCHUNK, RCHUNK)
    xs = _shuffle_sc(x, posm)
    tile_ids, group_ids, offs = _metadata(counts.reshape(E), N // T)
    prob_sorted = _grouped_mlp(xs, W1, b1, w2, b2, tile_ids, group_ids, offs)
    return _unsort_sc(prob_sorted, pos.reshape(NW, seg))


# speedup vs baseline: 1.3189x; 1.1105x over previous
import functools

import jax
import jax.numpy as jnp
from jax import lax
from jax.experimental import pallas as pl
from jax.experimental.pallas import tpu as pltpu
from jax.experimental.pallas import tpu_sc as plsc

_INTERPRET = False

T = 512
NW = 32
L = 16
RCHUNK = 64


def _pos_body(ch_ref, pos_ref, counts_ref):
    E = 16
    G, B = ch_ref.shape
    chm = ch_ref[...] % E
    tri_b = (jax.lax.broadcasted_iota(jnp.int32, (B, B), 0)
             < jax.lax.broadcasted_iota(jnp.int32, (B, B), 1)).astype(jnp.float32)
    tri_g = (jax.lax.broadcasted_iota(jnp.int32, (G, G), 1)
             < jax.lax.broadcasted_iota(jnp.int32, (G, G), 0)).astype(jnp.float32)
    iota_e = jax.lax.broadcasted_iota(jnp.int32, (1, E), 1)
    pos = jnp.zeros((G, B), jnp.float32)
    counts = jnp.zeros((1, E), jnp.int32)
    run_tot = jnp.float32(0.0)
    for e in range(E):
        m = (chm == e).astype(jnp.float32)
        rank = jnp.dot(m, tri_b, preferred_element_type=jnp.float32)
        cnt = jnp.sum(m, axis=1, keepdims=True)
        row_base = jnp.dot(tri_g, cnt, preferred_element_type=jnp.float32)
        tot = jnp.sum(cnt)
        pos = pos + m * (run_tot + row_base + rank)
        counts = jnp.where(iota_e == e, tot.astype(jnp.int32), counts)
        run_tot = run_tot + tot
    pos_ref[...] = pos.astype(jnp.int32)
    counts_ref[...] = counts


def _pos_tc(ch2d):
    G, B = ch2d.shape
    return pl.pallas_call(
        _pos_body,
        out_shape=[jax.ShapeDtypeStruct((G, B), jnp.int32),
                   jax.ShapeDtypeStruct((1, 16), jnp.int32)],
        interpret=_INTERPRET,
    )(ch2d)


def _shuffle_sc(xv, posm):
    N, DW = xv.shape
    seg = N // NW
    mesh = plsc.VectorSubcoreMesh(core_axis_name="c", subcore_axis_name="s")

    @functools.partial(
        pl.kernel, mesh=mesh,
        out_type=jax.ShapeDtypeStruct((N, DW), jnp.float32),
        scratch_types=[
            pltpu.VMEM((seg // RCHUNK, RCHUNK), jnp.int32),
            pltpu.VMEM((RCHUNK, DW), jnp.float32),
            pltpu.SemaphoreType.DMA,
        ],
    )
    def shuffle(xv_hbm, pos_hbm, xs_hbm, posm_v, rows_v, sem):
        w = lax.axis_index("s") * 2 + lax.axis_index("c")
        pltpu.sync_copy(pos_hbm.at[w], posm_v)
        for c in range(seg // RCHUNK):
            pltpu.sync_copy(xv_hbm.at[pl.ds(w * seg + c * RCHUNK, RCHUNK)], rows_v)
            pltpu.async_copy(rows_v, xs_hbm.at[posm_v.at[c]], sem).wait()

    return shuffle(xv, posm)


def _unsort_sc(prob_sorted, pos):
    N = prob_sorted.shape[0]
    seg = N // NW
    mesh = plsc.VectorSubcoreMesh(core_axis_name="c", subcore_axis_name="s")

    @functools.partial(
        pl.kernel, mesh=mesh,
        out_type=jax.ShapeDtypeStruct((NW, seg), jnp.float32),
        scratch_types=[
            pltpu.VMEM((seg,), jnp.int32),
            pltpu.VMEM((seg,), jnp.float32),
            pltpu.SemaphoreType.DMA,
        ],
    )
    def unsort(prob_hbm, pos_hbm, out_hbm, pos_v, vals_v, sem):
        w = lax.axis_index("s") * 2 + lax.axis_index("c")
        pltpu.sync_copy(pos_hbm.at[w], pos_v)
        pltpu.async_copy(prob_hbm.at[pos_v], vals_v, sem).wait()
        pltpu.sync_copy(vals_v, out_hbm.at[w])

    return unsort(prob_sorted, pos).reshape(N)


def _gmm_body(tile_ids, group_ids, offs, b2s,
              x_ref, w1_ref, b1_ref, w2_ref, out_ref):
    u = pl.program_id(0)
    e = group_ids[u]
    t = tile_ids[u]
    w1 = w1_ref[0].astype(jnp.bfloat16)
    b1r = b1_ref[0, 0, :][None, :]
    F = w1.shape[1]

    def _half(sl):
        h = jnp.tanh(
            jnp.dot(x_ref[sl, :].astype(jnp.bfloat16), w1,
                    preferred_element_type=jnp.float32) + b1r)
        acc = h[:, 0:128] * w2_ref[0, 0, 0:128][None, :]
        for k in range(1, F // 128):
            acc = acc + h[:, k * 128:(k + 1) * 128] * w2_ref[0, 0, k * 128:(k + 1) * 128][None, :]
        return jnp.sum(acc, axis=1, keepdims=True)

    H = T // 2
    logit = jnp.concatenate(
        [_half(pl.ds(0, H)), _half(pl.ds(H, H))], axis=0) + b2s[e]
    prob = jax.nn.softplus(logit)
    r = t * T + jax.lax.broadcasted_iota(jnp.int32, (T, 1), 0)
    mask = (r >= offs[e]) & (r < offs[e + 1])
    out_ref[...] = jnp.where(mask, prob, out_ref[...])


def _grouped_mlp(xs, W1, b1, w2, b2, tile_ids, group_ids, offs):
    N, D = xs.shape
    E, _, F = W1.shape
    U = tile_ids.shape[0]
    grid_spec = pltpu.PrefetchScalarGridSpec(
        num_scalar_prefetch=4,
        grid=(U,),
        in_specs=[
            pl.BlockSpec((T, D), lambda u, ti, gi, of, b2s: (ti[u], 0)),
            pl.BlockSpec((1, D, F), lambda u, ti, gi, of, b2s: (gi[u], 0, 0)),
            pl.BlockSpec((1, 1, F), lambda u, ti, gi, of, b2s: (gi[u], 0, 0)),
            pl.BlockSpec((1, 1, F), lambda u, ti, gi, of, b2s: (gi[u], 0, 0)),
        ],
        out_specs=pl.BlockSpec((T, 1), lambda u, ti, gi, of, b2s: (ti[u], 0)),
    )
    out = pl.pallas_call(
        _gmm_body,
        grid_spec=grid_spec,
        out_shape=jax.ShapeDtypeStruct((N, 1), jnp.float32),
        interpret=_INTERPRET,
    )(tile_ids, group_ids, offs, b2,
      xs, W1, b1.reshape(E, 1, F), w2.reshape(E, 1, F))
    return out.reshape(N)


def _metadata(counts, num_tiles):
    E = counts.shape[0]
    U = num_tiles + E - 1
    ends = jnp.cumsum(counts)
    starts = ends - counts
    offs = jnp.concatenate([jnp.zeros((1,), jnp.int32), ends.astype(jnp.int32)])
    start_tile = starts // T
    end_tile = jnp.maximum(ends - 1, 0) // T
    g = jnp.where(counts > 0, end_tile - start_tile + 1, 0)
    gcum = jnp.cumsum(g)
    total = gcum[-1]
    u = jnp.arange(U, dtype=jnp.int32)
    uc = jnp.minimum(u, total - 1)
    grp = jnp.searchsorted(gcum, uc, side="right").astype(jnp.int32)
    before = (gcum[grp] - g[grp]).astype(jnp.int32)
    tile = (start_tile[grp] + (uc - before)).astype(jnp.int32)
    return tile, grp, offs


def kernel(x, channel, W1, b1, w2, b2):
    N, D = x.shape
    E = W1.shape[0]
    seg = N // NW
    pos, counts = _pos_tc(channel.astype(jnp.int32).reshape(N // 128, 128))
    posm = pos.reshape(NW, seg // RCHUNK, RCHUNK)
    xs = _shuffle_sc(x, posm)
    tile_ids, group_ids, offs = _metadata(counts.reshape(E), N // T)
    prob_sorted = _grouped_mlp(xs, W1, b1, w2, b2, tile_ids, group_ids, offs)
    return _unsort_sc(prob_sorted, pos.reshape(NW, seg))
